# Initial kernel scaffold; baseline (speedup 1.0000x reference)
import jax
import jax.numpy as jnp
from jax.experimental import pallas as pl


def kernel(x, k_list):
    raise NotImplementedError("write your pallas kernel here")


# speedup vs baseline: 3.5244x; 3.5244x over previous
import jax
import jax.numpy as jnp
from jax import lax
from jax.experimental import pallas as pl
from jax.experimental.pallas import tpu as pltpu
from jax.experimental.pallas import tpu_sc as plsc

_SF = 2
_B, _C, _T = 64, 256, 4096
_R = 8
_CHUNK = _R * _T
_PAD = 16
_NC, _NS = 2, 16
_NW = _NC * _NS
_BPW = _B // _NW


def _body(x_hbm, k_hbm, out_hbm, buf, buf2, kvbuf):
    core = lax.axis_index("c")
    sub = lax.axis_index("s")
    wid = sub * _NC + core
    pltpu.sync_copy(k_hbm, kvbuf.at[pl.ds(0, _B)])
    lane = jax.lax.iota(jnp.int32, 16)
    for j in range(_BPW):
        b = wid * _BPW + j
        kk = kvbuf[pl.ds(b, 16)][0]
        cond_pos = (lane >= 8 - jnp.maximum(kk, 0)) & (lane < 8)
        cond_neg = (lane >= 8) & (lane < 8 - jnp.minimum(kk, 0))

        def chunk_body(ci, carry):
            start = (b * _C + ci * _R) * _T
            pltpu.sync_copy(x_hbm.at[pl.ds(start, _CHUNK)],
                            buf.at[pl.ds(_PAD, _CHUNK)])
            for i in range(_R + 1):
                pos = _PAD + i * _T
                w = buf[pl.ds(pos - 8, 16)]
                a = jnp.flip(buf[pl.ds(pos - 7, 16)])
                c = jnp.flip(buf[pl.ds(pos - 9, 16)])
                buf[pl.ds(pos - 8, 16)] = jnp.where(
                    cond_pos, a, jnp.where(cond_neg, c, w))

            src0 = _PAD - kk

            def copy_body(q, carry2):
                buf2[pl.ds(q * 16, 16)] = buf[pl.ds(src0 + q * 16, 16)]
                return carry2

            lax.fori_loop(0, _CHUNK // 16, copy_body, 0)
            pltpu.sync_copy(buf2, out_hbm.at[pl.ds(start, _CHUNK)])
            return carry

        lax.fori_loop(0, _C // _R, chunk_body, 0)


def kernel(x, k_list):
    k32 = k_list.astype(jnp.int32) - _SF
    xf = x.reshape(_B * _C * _T)
    mesh = plsc.VectorSubcoreMesh(core_axis_name="c", subcore_axis_name="s")
    f = pl.kernel(
        _body,
        mesh=mesh,
        out_type=jax.ShapeDtypeStruct((_B * _C * _T,), jnp.float32),
        scratch_types=[
            pltpu.VMEM((_PAD + _CHUNK + _PAD,), jnp.float32),
            pltpu.VMEM((_CHUNK,), jnp.float32),
            pltpu.VMEM((_B + 16,), jnp.int32),
        ],
    )
    out = f(xf, k32)
    return out.reshape(_B, _C, _T)


# speedup vs baseline: 6.2370x; 1.7697x over previous
import jax
import jax.numpy as jnp
from jax import lax
from jax.experimental import pallas as pl
from jax.experimental.pallas import tpu as pltpu
from jax.experimental.pallas import tpu_sc as plsc

_SF = 2
_B, _C, _T = 64, 256, 4096
_R = 8
_CHUNK = _R * _T
_PAD = 16
_NC, _NS = 2, 16
_NW = _NC * _NS
_BPW = _B // _NW


def _body(x_hbm, k_hbm, out_hbm, buf, buf2, kvbuf):
    core = lax.axis_index("c")
    sub = lax.axis_index("s")
    wid = sub * _NC + core
    pltpu.sync_copy(k_hbm, kvbuf.at[pl.ds(0, _B)])
    lane = jax.lax.iota(jnp.int32, 16)
    for j in range(_BPW):
        b = wid * _BPW + j
        kk = kvbuf[pl.ds(b, 16)][0]
        cond_pos = (lane >= 8 - jnp.maximum(kk, 0)) & (lane < 8)
        cond_neg = (lane >= 8) & (lane < 8 - jnp.minimum(kk, 0))

        def chunk_body(ci, carry):
            start = (b * _C + ci * _R) * _T
            pltpu.sync_copy(x_hbm.at[pl.ds(start, _CHUNK)],
                            buf.at[pl.ds(_PAD, _CHUNK)])
            for i in range(_R + 1):
                pos = _PAD + i * _T
                w = buf[pl.ds(pos - 8, 16)]
                a = jnp.flip(buf[pl.ds(pos - 7, 16)])
                c = jnp.flip(buf[pl.ds(pos - 9, 16)])
                buf[pl.ds(pos - 8, 16)] = jnp.where(
                    cond_pos, a, jnp.where(cond_neg, c, w))

            src0 = _PAD - kk

            @plsc.parallel_loop(0, _CHUNK, 16, unroll=8)
            def copy_body(q):
                buf2[pl.ds(q, 16)] = buf[pl.ds(src0 + q, 16)]
            pltpu.sync_copy(buf2, out_hbm.at[pl.ds(start, _CHUNK)])
            return carry

        lax.fori_loop(0, _C // _R, chunk_body, 0)


def kernel(x, k_list):
    k32 = k_list.astype(jnp.int32) - _SF
    xf = x.reshape(_B * _C * _T)
    mesh = plsc.VectorSubcoreMesh(core_axis_name="c", subcore_axis_name="s")
    f = pl.kernel(
        _body,
        mesh=mesh,
        out_type=jax.ShapeDtypeStruct((_B * _C * _T,), jnp.float32),
        scratch_types=[
            pltpu.VMEM((_PAD + _CHUNK + _PAD,), jnp.float32),
            pltpu.VMEM((_CHUNK,), jnp.float32),
            pltpu.VMEM((_B + 16,), jnp.int32),
        ],
    )
    out = f(xf, k32)
    return out.reshape(_B, _C, _T)


# speedup vs baseline: 6.2510x; 1.0022x over previous
import jax
import jax.numpy as jnp
from jax import lax
from jax.experimental import pallas as pl
from jax.experimental.pallas import tpu as pltpu
from jax.experimental.pallas import tpu_sc as plsc

_SF = 2
_B, _C, _T = 64, 256, 4096
_R = 8
_CHUNK = _R * _T
_PAD = 16
_NC, _NS = 2, 16
_NW = _NC * _NS
_BPW = _B // _NW


def _body(x_hbm, k_hbm, out_hbm, buf, buf2, kvbuf):
    core = lax.axis_index("c")
    sub = lax.axis_index("s")
    wid = sub * _NC + core
    pltpu.sync_copy(k_hbm, kvbuf.at[pl.ds(0, _B)])
    lane = jax.lax.iota(jnp.int32, 16)
    for j in range(_BPW):
        b = wid * _BPW + j
        kk = kvbuf[pl.ds(b, 16)][0]
        cond_pos = (lane >= 8 - jnp.maximum(kk, 0)) & (lane < 8)
        cond_neg = (lane >= 8) & (lane < 8 - jnp.minimum(kk, 0))

        def chunk_body(ci, carry):
            start = (b * _C + ci * _R) * _T
            pltpu.sync_copy(x_hbm.at[pl.ds(start, _CHUNK)],
                            buf.at[pl.ds(_PAD, _CHUNK)])
            for i in range(_R + 1):
                pos = _PAD + i * _T
                w = buf[pl.ds(pos - 8, 16)]
                a = jnp.flip(buf[pl.ds(pos - 7, 16)])
                c = jnp.flip(buf[pl.ds(pos - 9, 16)])
                buf[pl.ds(pos - 8, 16)] = jnp.where(
                    cond_pos, a, jnp.where(cond_neg, c, w))

            src0 = _PAD - kk

            @plsc.parallel_loop(0, _CHUNK, 128, unroll=4)
            def copy_body(q):
                for u in range(8):
                    buf2[pl.ds(q + u * 16, 16)] = buf[
                        pl.ds(src0 + q + u * 16, 16)]
            pltpu.sync_copy(buf2, out_hbm.at[pl.ds(start, _CHUNK)])
            return carry

        lax.fori_loop(0, _C // _R, chunk_body, 0)


def kernel(x, k_list):
    k32 = k_list.astype(jnp.int32) - _SF
    xf = x.reshape(_B * _C * _T)
    mesh = plsc.VectorSubcoreMesh(core_axis_name="c", subcore_axis_name="s")
    f = pl.kernel(
        _body,
        mesh=mesh,
        out_type=jax.ShapeDtypeStruct((_B * _C * _T,), jnp.float32),
        scratch_types=[
            pltpu.VMEM((_PAD + _CHUNK + _PAD,), jnp.float32),
            pltpu.VMEM((_CHUNK,), jnp.float32),
            pltpu.VMEM((_B + 16,), jnp.int32),
        ],
    )
    out = f(xf, k32)
    return out.reshape(_B, _C, _T)


# speedup vs baseline: 7.3127x; 1.1699x over previous
import jax
import jax.numpy as jnp
from jax import lax
from jax.experimental import pallas as pl
from jax.experimental.pallas import tpu as pltpu
from jax.experimental.pallas import tpu_sc as plsc

_SF = 2
_B, _C, _T = 64, 256, 4096
_R = 4
_CHUNK = _R * _T
_PADF = 16
_PADB = 16
_NC, _NS = 2, 16
_NW = _NC * _NS
_BPW = _B // _NW
_NCH = _C // _R


def _body(x_hbm, k_hbm, out_hbm,
          buf_a, buf_b, buf2_a, buf2_b, kvbuf,
          lsem_a, lsem_b, ssem_a, ssem_b):
    core = lax.axis_index("c")
    sub = lax.axis_index("s")
    wid = sub * _NC + core
    pltpu.sync_copy(k_hbm, kvbuf.at[pl.ds(0, _B)])
    lane = jax.lax.iota(jnp.int32, 16)
    for j in range(_BPW):
        b = wid * _BPW + j
        kk = kvbuf[pl.ds(b, 16)][0]
        cond_pos = (lane >= 8 - jnp.maximum(kk, 0)) & (lane < 8)
        cond_neg = (lane >= 8) & (lane < 8 - jnp.minimum(kk, 0))
        base = b * _C * _T

        def load_dma(ci, buf_x, sem):
            return pltpu.make_async_copy(
                x_hbm.at[pl.ds(base + ci * _CHUNK, _CHUNK)],
                buf_x.at[pl.ds(_PADF, _CHUNK)], sem)

        def store_dma(ci, buf2_x, sem):
            return pltpu.make_async_copy(
                buf2_x, out_hbm.at[pl.ds(base + ci * _CHUNK, _CHUNK)], sem)

        load_dma(0, buf_a, lsem_a).start()
        load_dma(1, buf_b, lsem_b).start()

        def pair_body(i2, carry):
            for ph, buf_x, buf2_x, lsem, ssem in (
                    (0, buf_a, buf2_a, lsem_a, ssem_a),
                    (1, buf_b, buf2_b, lsem_b, ssem_b)):
                ci = 2 * i2 + ph
                load_dma(ci, buf_x, lsem).wait()
                for i in range(_R + 1):
                    pos = _PADF + i * _T
                    w = buf_x[pl.ds(pos - 8, 16)]
                    a = jnp.flip(buf_x[pl.ds(pos - 7, 16)])
                    c = jnp.flip(buf_x[pl.ds(pos - 9, 16)])
                    buf_x[pl.ds(pos - 8, 16)] = jnp.where(
                        cond_pos, a, jnp.where(cond_neg, c, w))

                @pl.when(i2 > 0)
                def _():
                    store_dma(ci - 2, buf2_x, ssem).wait()

                src0 = _PADF - kk

                @plsc.parallel_loop(0, _CHUNK, 128, unroll=4)
                def copy_body(q):
                    for u in range(8):
                        buf2_x[pl.ds(q + u * 16, 16)] = buf_x[
                            pl.ds(src0 + q + u * 16, 16)]

                store_dma(ci, buf2_x, ssem).start()

                @pl.when(ci + 2 < _NCH)
                def _():
                    load_dma(ci + 2, buf_x, lsem).start()
            return carry

        lax.fori_loop(0, _NCH // 2, pair_body, 0)
        store_dma(_NCH - 2, buf2_a, ssem_a).wait()
        store_dma(_NCH - 1, buf2_b, ssem_b).wait()


def kernel(x, k_list):
    k32 = k_list.astype(jnp.int32) - _SF
    xf = x.reshape(_B * _C * _T)
    mesh = plsc.VectorSubcoreMesh(core_axis_name="c", subcore_axis_name="s")
    f = pl.kernel(
        _body,
        mesh=mesh,
        out_type=jax.ShapeDtypeStruct((_B * _C * _T,), jnp.float32),
        scratch_types=[
            pltpu.VMEM((_PADF + _CHUNK + _PADB,), jnp.float32),
            pltpu.VMEM((_PADF + _CHUNK + _PADB,), jnp.float32),
            pltpu.VMEM((_CHUNK,), jnp.float32),
            pltpu.VMEM((_CHUNK,), jnp.float32),
            pltpu.VMEM((_B + 16,), jnp.int32),
            pltpu.SemaphoreType.DMA,
            pltpu.SemaphoreType.DMA,
            pltpu.SemaphoreType.DMA,
            pltpu.SemaphoreType.DMA,
        ],
    )
    out = f(xf, k32)
    return out.reshape(_B, _C, _T)


# speedup vs baseline: 10.6818x; 1.4607x over previous
import jax
import jax.numpy as jnp
from jax import lax
from jax.experimental import pallas as pl
from jax.experimental.pallas import tpu as pltpu
from jax.experimental.pallas import tpu_sc as plsc

_SF = 2
_B, _C, _T = 64, 256, 4096
_R = 8
_CHUNK = _R * _T
_PADF = 16
_PADB = 16
_NB = _T // 128
_NC, _NS = 2, 16
_NW = _NC * _NS
_BPW = _B // _NW
_NCH = _C // _R


def _body(x_hbm, k_hbm, out_hbm,
          buf, buf2_a, buf2_b, kvbuf, ssem_a, ssem_b):
    core = lax.axis_index("c")
    sub = lax.axis_index("s")
    wid = sub * _NC + core
    pltpu.sync_copy(k_hbm, kvbuf.at[pl.ds(0, _B)])
    lane = jax.lax.iota(jnp.int32, 16)
    for j in range(_BPW):
        b = wid * _BPW + j
        kk = kvbuf[pl.ds(b, 16)][0]
        cond_pos = (lane >= 8 - jnp.maximum(kk, 0)) & (lane < 8)
        cond_neg = (lane >= 8) & (lane < 8 - jnp.minimum(kk, 0))
        base = b * _C * _T

        def store_dma(ci, buf2_x, sem):
            return pltpu.make_async_copy(
                buf2_x, out_hbm.at[b, pl.ds(ci * _R, _R), :], sem)

        def pair_body(i2, carry):
            for ph, buf2_x, ssem in ((0, buf2_a, ssem_a),
                                     (1, buf2_b, ssem_b)):
                ci = 2 * i2 + ph
                pltpu.sync_copy(x_hbm.at[pl.ds(base + ci * _CHUNK, _CHUNK)],
                                buf.at[pl.ds(_PADF, _CHUNK)])
                for i in range(_R + 1):
                    pos = _PADF + i * _T
                    w = buf[pl.ds(pos - 8, 16)]
                    a = jnp.flip(buf[pl.ds(pos - 7, 16)])
                    c = jnp.flip(buf[pl.ds(pos - 9, 16)])
                    buf[pl.ds(pos - 8, 16)] = jnp.where(
                        cond_pos, a, jnp.where(cond_neg, c, w))

                @pl.when(i2 > 0)
                def _():
                    store_dma(ci - 2, buf2_x, ssem).wait()

                src00 = _PADF - kk

                @plsc.parallel_loop(0, _R * _NB, 1, unroll=2)
                def copy_body(m):
                    rr = m >> 5
                    col0 = (m & 31) * 128
                    s0 = src00 + rr * _T + col0
                    for u in range(8):
                        buf2_x[rr, pl.ds(col0 + u * 16, 16)] = buf[
                            pl.ds(s0 + u * 16, 16)]

                store_dma(ci, buf2_x, ssem).start()
            return carry

        lax.fori_loop(0, _NCH // 2, pair_body, 0)
        store_dma(_NCH - 2, buf2_a, ssem_a).wait()
        store_dma(_NCH - 1, buf2_b, ssem_b).wait()


def kernel(x, k_list):
    k32 = k_list.astype(jnp.int32) - _SF
    xf = x.reshape(_B * _C * _T)
    mesh = plsc.VectorSubcoreMesh(core_axis_name="c", subcore_axis_name="s")
    f = pl.kernel(
        _body,
        mesh=mesh,
        out_type=jax.ShapeDtypeStruct((_B, _C, _T), jnp.float32),
        compiler_params=pltpu.CompilerParams(use_tc_tiling_on_sc=True),
        scratch_types=[
            pltpu.VMEM((_PADF + _CHUNK + _PADB,), jnp.float32),
            pltpu.VMEM((_R, _T), jnp.float32),
            pltpu.VMEM((_R, _T), jnp.float32),
            pltpu.VMEM((_B + 16,), jnp.int32),
            pltpu.SemaphoreType.DMA,
            pltpu.SemaphoreType.DMA,
        ],
    )
    return f(xf, k32)


# speedup vs baseline: 13.7088x; 1.2834x over previous
import jax
import jax.numpy as jnp
from jax import lax
from jax.experimental import pallas as pl
from jax.experimental.pallas import tpu as pltpu
from jax.experimental.pallas import tpu_sc as plsc

_SF = 2
_B, _C, _T = 64, 256, 4096
_R = 8
_W = _T // 2
_PADF = 16
_PADB = 16
_NC, _NS = 2, 16
_NW = _NC * _NS
_BPW = _B // _NW
_NCH = _C // _R


def _body(x_hbm, k_hbm, out_hbm,
          bufin, buf, buf2_a, buf2_b, kvbuf, ssem_a, ssem_b):
    core = lax.axis_index("c")
    sub = lax.axis_index("s")
    wid = sub * _NC + core
    pltpu.sync_copy(k_hbm, kvbuf.at[pl.ds(0, _B)])
    lane = jax.lax.iota(jnp.int32, 16)
    for j in range(_BPW):
        b = wid * _BPW + j
        kk = kvbuf[pl.ds(b, 16)][0]
        cond_pos = (lane >= 8 - jnp.maximum(kk, 0)) & (lane < 8)
        cond_neg = (lane >= 8) & (lane < 8 - jnp.minimum(kk, 0))
        seam_lo = lane < -jnp.minimum(kk, 0)
        seam_hi = lane >= 16 - jnp.maximum(kk, 0)

        def store_dma(ci, buf2_x, sem):
            return pltpu.make_async_copy(
                buf2_x, out_hbm.at[b, pl.ds(ci * _R, _R), :], sem)

        def pair_body(i2, carry):
            for ph, buf2_x, ssem in ((0, buf2_a, ssem_a),
                                     (1, buf2_b, ssem_b)):
                ci = 2 * i2 + ph
                pltpu.sync_copy(x_hbm.at[b, pl.ds(ci * _R, _R), :], bufin)

                @pl.when(i2 > 0)
                def _():
                    store_dma(ci - 2, buf2_x, ssem).wait()

                for half in range(2):
                    h0 = half * _W

                    @plsc.parallel_loop(0, _R * _W // 128, 1, unroll=2)
                    def detile(m):
                        rr = m >> 4
                        col0 = (m & 15) * 128
                        for u in range(8):
                            buf[pl.ds(_PADF + rr * _W + col0 + u * 16,
                                      16)] = bufin[
                                rr, pl.ds(h0 + col0 + u * 16, 16)]

                    for i in range(_R + 1):
                        pos = _PADF + i * _W
                        if half == 0:
                            w1 = buf[pl.ds(pos - 8, 16)]
                            a = jnp.flip(buf[pl.ds(pos - 7, 16)])
                            buf[pl.ds(pos - 8, 16)] = jnp.where(
                                cond_pos, a, w1)
                            w2 = buf[pl.ds(pos, 16)]
                            hi = bufin[max(i - 1, 0), pl.ds(_W, 16)]
                            buf[pl.ds(pos, 16)] = jnp.where(
                                seam_lo, hi, w2)
                        else:
                            w1 = buf[pl.ds(pos - 16, 16)]
                            lo = bufin[min(i, _R - 1), pl.ds(_W - 16, 16)]
                            buf[pl.ds(pos - 16, 16)] = jnp.where(
                                seam_hi, lo, w1)
                            w2 = buf[pl.ds(pos - 8, 16)]
                            c = jnp.flip(buf[pl.ds(pos - 9, 16)])
                            buf[pl.ds(pos - 8, 16)] = jnp.where(
                                cond_neg, c, w2)

                    src00 = _PADF - kk

                    @plsc.parallel_loop(0, _R * _W // 128, 1, unroll=2)
                    def retile(m):
                        rr = m >> 4
                        col0 = (m & 15) * 128
                        s0 = src00 + rr * _W + col0
                        for u in range(8):
                            buf2_x[rr, pl.ds(h0 + col0 + u * 16, 16)] = buf[
                                pl.ds(s0 + u * 16, 16)]

                store_dma(ci, buf2_x, ssem).start()
            return carry

        lax.fori_loop(0, _NCH // 2, pair_body, 0)
        store_dma(_NCH - 2, buf2_a, ssem_a).wait()
        store_dma(_NCH - 1, buf2_b, ssem_b).wait()


def kernel(x, k_list):
    k32 = k_list.astype(jnp.int32) - _SF
    mesh = plsc.VectorSubcoreMesh(core_axis_name="c", subcore_axis_name="s")
    f = pl.kernel(
        _body,
        mesh=mesh,
        out_type=jax.ShapeDtypeStruct((_B, _C, _T), jnp.float32),
        compiler_params=pltpu.CompilerParams(use_tc_tiling_on_sc=True),
        scratch_types=[
            pltpu.VMEM((_R, _T), jnp.float32),
            pltpu.VMEM((_PADF + _R * _W + _PADB,), jnp.float32),
            pltpu.VMEM((_R, _T), jnp.float32),
            pltpu.VMEM((_R, _T), jnp.float32),
            pltpu.VMEM((_B + 16,), jnp.int32),
            pltpu.SemaphoreType.DMA,
            pltpu.SemaphoreType.DMA,
        ],
    )
    return f(x, k32)


# speedup vs baseline: 20.7422x; 1.5131x over previous
import jax
import jax.numpy as jnp
from jax import lax
from jax.experimental import pallas as pl
from jax.experimental.pallas import tpu as pltpu
from jax.experimental.pallas import tpu_sc as plsc

_SF = 2
_B, _C, _T = 64, 256, 4096
_R = 8
_W = _T // 2
_NBLK = _W // 128
_PADF = 16
_PADB = 16
_NC, _NS = 2, 16
_NW = _NC * _NS
_BPW = _B // _NW
_NCH = _C // _R


def _body(x_hbm, k_hbm, out_hbm,
          bufin0, bufin1, mini15, mini16, buf, buf2_a, buf2_b, kvbuf,
          lsem0, lsem1, msem, ssem_a, ssem_b):
    core = lax.axis_index("c")
    sub = lax.axis_index("s")
    wid = sub * _NC + core
    pltpu.sync_copy(k_hbm, kvbuf.at[pl.ds(0, _B)])
    lane = jax.lax.iota(jnp.int32, 16)
    for j in range(_BPW):
        b = wid * _BPW + j
        kk = kvbuf[pl.ds(b, 16)][0]
        cond_pos = (lane >= 8 - jnp.maximum(kk, 0)) & (lane < 8)
        cond_neg = (lane >= 8) & (lane < 8 - jnp.minimum(kk, 0))
        seam_lo = lane < -jnp.minimum(kk, 0)
        seam_hi = lane >= 16 - jnp.maximum(kk, 0)

        def load_half(ci, half, buf_x, sem):
            return pltpu.make_async_copy(
                x_hbm.at[b, pl.ds(ci * _R, _R), pl.ds(half * _W, _W)],
                buf_x, sem)

        def load_mini(ci, blk, mini_x, sem):
            return pltpu.make_async_copy(
                x_hbm.at[b, pl.ds(ci * _R, _R), pl.ds(blk * 128, 128)],
                mini_x, sem)

        def store_dma(ci, buf2_x, sem):
            return pltpu.make_async_copy(
                buf2_x, out_hbm.at[b, pl.ds(ci * _R, _R), :], sem)

        load_half(0, 0, bufin0, lsem0).start()
        load_half(0, 1, bufin1, lsem1).start()
        load_mini(0, _NBLK - 1, mini15, msem).start()
        load_mini(0, _NBLK, mini16, msem).start()

        def pair_body(i2, carry):
            for ph, buf2_x, ssem in ((0, buf2_a, ssem_a),
                                     (1, buf2_b, ssem_b)):
                ci = 2 * i2 + ph
                load_mini(ci, _NBLK - 1, mini15, msem).wait()
                load_mini(ci, _NBLK, mini16, msem).wait()
                for half in range(2):
                    h0 = half * _W
                    bufin = bufin0 if half == 0 else bufin1
                    lsem = lsem0 if half == 0 else lsem1
                    load_half(ci, half, bufin, lsem).wait()

                    @plsc.parallel_loop(0, _R * _NBLK, 1, unroll=2)
                    def detile(m):
                        rr = m >> 4
                        col0 = (m & 15) * 128
                        for u in range(8):
                            buf[pl.ds(_PADF + rr * _W + col0 + u * 16,
                                      16)] = bufin[
                                rr, pl.ds(col0 + u * 16, 16)]

                    @pl.when(ci + 1 < _NCH)
                    def _():
                        load_half(ci + 1, half, bufin, lsem).start()

                    for i in range(_R + 1):
                        pos = _PADF + i * _W
                        if half == 0:
                            w1 = buf[pl.ds(pos - 8, 16)]
                            a = jnp.flip(buf[pl.ds(pos - 7, 16)])
                            buf[pl.ds(pos - 8, 16)] = jnp.where(
                                cond_pos, a, w1)
                            w2 = buf[pl.ds(pos, 16)]
                            hi = mini16[max(i - 1, 0), pl.ds(0, 16)]
                            buf[pl.ds(pos, 16)] = jnp.where(
                                seam_lo, hi, w2)
                        else:
                            w1 = buf[pl.ds(pos - 16, 16)]
                            lo = mini15[min(i, _R - 1), pl.ds(112, 16)]
                            buf[pl.ds(pos - 16, 16)] = jnp.where(
                                seam_hi, lo, w1)
                            w2 = buf[pl.ds(pos - 8, 16)]
                            c = jnp.flip(buf[pl.ds(pos - 9, 16)])
                            buf[pl.ds(pos - 8, 16)] = jnp.where(
                                cond_neg, c, w2)

                    if half == 1:
                        @pl.when(ci + 1 < _NCH)
                        def _():
                            load_mini(ci + 1, _NBLK - 1, mini15,
                                      msem).start()
                            load_mini(ci + 1, _NBLK, mini16, msem).start()

                    if half == 0:
                        @pl.when(i2 > 0)
                        def _():
                            store_dma(ci - 2, buf2_x, ssem).wait()

                    src00 = _PADF - kk

                    @plsc.parallel_loop(0, _R * _NBLK, 1, unroll=2)
                    def retile(m):
                        rr = m >> 4
                        col0 = (m & 15) * 128
                        s0 = src00 + rr * _W + col0
                        for u in range(8):
                            buf2_x[rr, pl.ds(h0 + col0 + u * 16,
                                             16)] = buf[
                                pl.ds(s0 + u * 16, 16)]

                store_dma(ci, buf2_x, ssem).start()
            return carry

        lax.fori_loop(0, _NCH // 2, pair_body, 0)
        store_dma(_NCH - 2, buf2_a, ssem_a).wait()
        store_dma(_NCH - 1, buf2_b, ssem_b).wait()


def kernel(x, k_list):
    k32 = k_list.astype(jnp.int32) - _SF
    mesh = plsc.VectorSubcoreMesh(core_axis_name="c", subcore_axis_name="s")
    f = pl.kernel(
        _body,
        mesh=mesh,
        out_type=jax.ShapeDtypeStruct((_B, _C, _T), jnp.float32),
        compiler_params=pltpu.CompilerParams(use_tc_tiling_on_sc=True),
        scratch_types=[
            pltpu.VMEM((_R, _W), jnp.float32),
            pltpu.VMEM((_R, _W), jnp.float32),
            pltpu.VMEM((_R, 128), jnp.float32),
            pltpu.VMEM((_R, 128), jnp.float32),
            pltpu.VMEM((_PADF + _R * _W + _PADB,), jnp.float32),
            pltpu.VMEM((_R, _T), jnp.float32),
            pltpu.VMEM((_R, _T), jnp.float32),
            pltpu.VMEM((_B + 16,), jnp.int32),
            pltpu.SemaphoreType.DMA,
            pltpu.SemaphoreType.DMA,
            pltpu.SemaphoreType.DMA,
            pltpu.SemaphoreType.DMA,
            pltpu.SemaphoreType.DMA,
        ],
    )
    return f(x, k32)


# speedup vs baseline: 20.7657x; 1.0011x over previous
import jax
import jax.numpy as jnp
from jax import lax
from jax.experimental import pallas as pl
from jax.experimental.pallas import tpu as pltpu
from jax.experimental.pallas import tpu_sc as plsc

_SF = 2
_B, _C, _T = 64, 256, 4096
_R = 8
_W = _T // 2
_NBLK = _W // 128
_PADF = 16
_PADB = 16
_NC, _NS = 2, 16
_NW = _NC * _NS
_BPW = _B // _NW
_NCH = _C // _R


def _body(x_hbm, k_hbm, out_hbm,
          bufin0, bufin1, mini15, mini16, buf, buf2_a, buf2_b, kvbuf,
          lsem0, lsem1, msem, ssem_a, ssem_b):
    core = lax.axis_index("c")
    sub = lax.axis_index("s")
    wid = sub * _NC + core
    pltpu.sync_copy(k_hbm, kvbuf.at[pl.ds(0, _B)])
    lane = jax.lax.iota(jnp.int32, 16)
    for j in range(_BPW):
        b = wid * _BPW + j
        kk = kvbuf[pl.ds(b, 16)][0]
        cond_pos = (lane >= 8 - jnp.maximum(kk, 0)) & (lane < 8)
        cond_neg = (lane >= 8) & (lane < 8 - jnp.minimum(kk, 0))
        seam_lo = lane < -jnp.minimum(kk, 0)
        seam_hi = lane >= 16 - jnp.maximum(kk, 0)

        def load_half(ci, half, buf_x, sem):
            return pltpu.make_async_copy(
                x_hbm.at[b, pl.ds(ci * _R, _R), pl.ds(half * _W, _W)],
                buf_x, sem)

        def load_mini(ci, blk, mini_x, sem):
            return pltpu.make_async_copy(
                x_hbm.at[b, pl.ds(ci * _R, _R), pl.ds(blk * 128, 128)],
                mini_x, sem)

        def store_dma(ci, buf2_x, sem):
            return pltpu.make_async_copy(
                buf2_x, out_hbm.at[b, pl.ds(ci * _R, _R), :], sem)

        load_half(0, 0, bufin0, lsem0).start()
        load_half(0, 1, bufin1, lsem1).start()
        load_mini(0, _NBLK - 1, mini15, msem).start()
        load_mini(0, _NBLK, mini16, msem).start()

        def pair_body(i2, carry):
            for ph, buf2_x, ssem in ((0, buf2_a, ssem_a),
                                     (1, buf2_b, ssem_b)):
                ci = 2 * i2 + ph
                load_mini(ci, _NBLK - 1, mini15, msem).wait()
                load_mini(ci, _NBLK, mini16, msem).wait()
                for half in range(2):
                    h0 = half * _W
                    bufin = bufin0 if half == 0 else bufin1
                    lsem = lsem0 if half == 0 else lsem1
                    load_half(ci, half, bufin, lsem).wait()

                    @plsc.parallel_loop(0, _R * _NBLK, 1, unroll=4)
                    def detile(m):
                        rr = m >> 4
                        col0 = (m & 15) * 128
                        for u in range(8):
                            buf[pl.ds(_PADF + rr * _W + col0 + u * 16,
                                      16)] = bufin[
                                rr, pl.ds(col0 + u * 16, 16)]

                    @pl.when(ci + 1 < _NCH)
                    def _():
                        load_half(ci + 1, half, bufin, lsem).start()

                    for i in range(_R + 1):
                        pos = _PADF + i * _W
                        if half == 0:
                            w1 = buf[pl.ds(pos - 8, 16)]
                            a = jnp.flip(buf[pl.ds(pos - 7, 16)])
                            buf[pl.ds(pos - 8, 16)] = jnp.where(
                                cond_pos, a, w1)
                            w2 = buf[pl.ds(pos, 16)]
                            hi = mini16[max(i - 1, 0), pl.ds(0, 16)]
                            buf[pl.ds(pos, 16)] = jnp.where(
                                seam_lo, hi, w2)
                        else:
                            w1 = buf[pl.ds(pos - 16, 16)]
                            lo = mini15[min(i, _R - 1), pl.ds(112, 16)]
                            buf[pl.ds(pos - 16, 16)] = jnp.where(
                                seam_hi, lo, w1)
                            w2 = buf[pl.ds(pos - 8, 16)]
                            c = jnp.flip(buf[pl.ds(pos - 9, 16)])
                            buf[pl.ds(pos - 8, 16)] = jnp.where(
                                cond_neg, c, w2)

                    if half == 1:
                        @pl.when(ci + 1 < _NCH)
                        def _():
                            load_mini(ci + 1, _NBLK - 1, mini15,
                                      msem).start()
                            load_mini(ci + 1, _NBLK, mini16, msem).start()

                    if half == 0:
                        @pl.when(i2 > 0)
                        def _():
                            store_dma(ci - 2, buf2_x, ssem).wait()

                    src00 = _PADF - kk

                    @plsc.parallel_loop(0, _R * _NBLK, 1, unroll=4)
                    def retile(m):
                        rr = m >> 4
                        col0 = (m & 15) * 128
                        s0 = src00 + rr * _W + col0
                        for u in range(8):
                            buf2_x[rr, pl.ds(h0 + col0 + u * 16,
                                             16)] = buf[
                                pl.ds(s0 + u * 16, 16)]

                store_dma(ci, buf2_x, ssem).start()
            return carry

        lax.fori_loop(0, _NCH // 2, pair_body, 0)
        store_dma(_NCH - 2, buf2_a, ssem_a).wait()
        store_dma(_NCH - 1, buf2_b, ssem_b).wait()


def kernel(x, k_list):
    k32 = k_list.astype(jnp.int32) - _SF
    mesh = plsc.VectorSubcoreMesh(core_axis_name="c", subcore_axis_name="s")
    f = pl.kernel(
        _body,
        mesh=mesh,
        out_type=jax.ShapeDtypeStruct((_B, _C, _T), jnp.float32),
        compiler_params=pltpu.CompilerParams(use_tc_tiling_on_sc=True),
        scratch_types=[
            pltpu.VMEM((_R, _W), jnp.float32),
            pltpu.VMEM((_R, _W), jnp.float32),
            pltpu.VMEM((_R, 128), jnp.float32),
            pltpu.VMEM((_R, 128), jnp.float32),
            pltpu.VMEM((_PADF + _R * _W + _PADB,), jnp.float32),
            pltpu.VMEM((_R, _T), jnp.float32),
            pltpu.VMEM((_R, _T), jnp.float32),
            pltpu.VMEM((_B + 16,), jnp.int32),
            pltpu.SemaphoreType.DMA,
            pltpu.SemaphoreType.DMA,
            pltpu.SemaphoreType.DMA,
            pltpu.SemaphoreType.DMA,
            pltpu.SemaphoreType.DMA,
        ],
    )
    return f(x, k32)
